# baseline (speedup 1.0000x reference)
import functools

import jax
import jax.numpy as jnp
from jax import lax
from jax.experimental import pallas as pl
from jax.experimental.pallas import tpu as pltpu
from jax.experimental.pallas import tpu_sc as plsc

NUM_EMBEDDINGS = 1000000
EMBED_DIM = 64
BATCH, SEQ = 16384, 50
TOTAL = BATCH * SEQ

NC, NS = 2, 16
NW = NC * NS
PER_W = TOTAL // NW
CHUNK = 256
NCHUNK = PER_W // CHUNK
NBUF = 4
K = 2


def _gather_body(idx_hbm, tbl_hbm, out_hbm, idx_all, rows_v, gsem, osem):
    wid = lax.axis_index("s") * NC + lax.axis_index("c")
    base = wid * PER_W
    pltpu.sync_copy(idx_hbm.at[pl.ds(base, PER_W)], idx_all)

    def gather_desc(c):
        b = lax.rem(c, NBUF)
        return pltpu.make_async_copy(
            tbl_hbm.at[idx_all.at[pl.ds(c * CHUNK, CHUNK)]],
            rows_v.at[b],
            gsem.at[b],
        )

    def out_desc(c):
        b = lax.rem(c, NBUF)
        return pltpu.make_async_copy(
            rows_v.at[b],
            out_hbm.at[pl.ds(base + c * CHUNK, CHUNK)],
            osem.at[b],
        )

    for c in range(K):
        gather_desc(c).start()

    def step(c, carry):
        gather_desc(c).wait()
        out_desc(c).start()

        @pl.when(c + K >= NBUF)
        def _wait_buf():
            out_desc(c + K - NBUF).wait()

        gather_desc(c + K).start()
        return carry

    lax.fori_loop(0, NCHUNK - K, step, 0)

    for c in range(NCHUNK - K, NCHUNK):
        gather_desc(c).wait()
        out_desc(c).start()
    for c in range(NCHUNK - NBUF, NCHUNK):
        out_desc(c).wait()


BPG = 8
GROUPS = BATCH // (NW * BPG)
RBUF = 2


def _relayout_body(rows_hbm, out_hbm, stage_v, sem):
    wid = lax.axis_index("s") * NC + lax.axis_index("c")
    nb_per_w = BATCH // NW
    b0 = wid * nb_per_w

    def stage_desc(g, r):
        b = b0 + g * BPG
        return pltpu.make_async_copy(
            rows_hbm.at[pl.ds(b * SEQ, BPG * SEQ)], stage_v.at[r], sem.at[r]
        )

    stage_desc(0, 0).start()

    def step(g, carry):
        r = lax.rem(g, RBUF)

        @pl.when(g + 1 < GROUPS)
        def _prefetch():
            stage_desc(g + 1, lax.rem(g + 1, RBUF)).start()

        stage_desc(g, r).wait()
        bg = b0 + g * BPG

        def inner(i, carry2):
            pltpu.sync_copy(
                stage_v.at[r].at[pl.ds(i * SEQ, SEQ)], out_hbm.at[bg + i]
            )
            return carry2

        lax.fori_loop(0, BPG, inner, 0)
        return carry

    lax.fori_loop(0, GROUPS, step, 0)


@jax.jit
def _embed(idx_flat, weight):
    k1 = pl.kernel(
        _gather_body,
        out_type=jax.ShapeDtypeStruct((TOTAL, EMBED_DIM), jnp.float32),
        mesh=plsc.VectorSubcoreMesh(core_axis_name="c", subcore_axis_name="s"),
        scratch_types=[
            pltpu.VMEM((PER_W,), jnp.int32),
            pltpu.VMEM((NBUF, CHUNK, EMBED_DIM), jnp.float32),
            pltpu.SemaphoreType.DMA((NBUF,)),
            pltpu.SemaphoreType.DMA((NBUF,)),
        ],
        compiler_params=pltpu.CompilerParams(use_tc_tiling_on_sc=False),
    )
    rows = k1(idx_flat, weight)
    k2 = pl.kernel(
        _relayout_body,
        out_type=jax.ShapeDtypeStruct((BATCH, SEQ, EMBED_DIM), jnp.float32),
        mesh=plsc.VectorSubcoreMesh(core_axis_name="c", subcore_axis_name="s"),
        scratch_types=[
            pltpu.VMEM((RBUF, BPG * SEQ, EMBED_DIM), jnp.float32),
            pltpu.SemaphoreType.DMA((RBUF,)),
        ],
        compiler_params=pltpu.CompilerParams(use_tc_tiling_on_sc=True),
    )
    return k2(rows)


def kernel(input_, weight):
    idx_flat = input_.astype(jnp.int32).reshape(TOTAL)
    return _embed(idx_flat, weight)
